# Initial kernel scaffold; baseline (speedup 1.0000x reference)
import jax
import jax.numpy as jnp
from jax.experimental import pallas as pl


def kernel(features1, edge_index1, features2, edge_index2, features3, edge_index3, params):
    raise NotImplementedError("write your pallas kernel here")


# speedup vs baseline: 1.6774x; 1.6774x over previous
import functools

import jax
import jax.numpy as jnp
from jax import lax
from jax.experimental import pallas as pl
from jax.experimental.pallas import tpu as pltpu
from jax.experimental.pallas import tpu_sc as plsc

N = 10000
D = 128
E = 320000

NC = 2
NS = 16
NW = NC * NS
CHUNK = 128
EPW = 10240
NCHUNK = EPW // CHUNK
E_PAD = NW * EPW
ACC_ROWS = 10240
ZR = 64

_mesh = plsc.VectorSubcoreMesh(core_axis_name="c", subcore_axis_name="s")


def _zero_fill(zbuf, rows, cols):
    z = jnp.zeros((16,), jnp.float32)
    for i in range(rows):
        for k in range(cols // 16):
            zbuf[i, pl.ds(16 * k, 16)] = z


def _make_propagate(xd):
    @functools.partial(
        pl.kernel,
        out_type=jax.ShapeDtypeStruct((NC, N, xd), jnp.float32),
        mesh=_mesh,
        scratch_types=[
            pltpu.VMEM((NCHUNK, CHUNK), jnp.int32),
            pltpu.VMEM((2, CHUNK), jnp.int32),
            pltpu.VMEM((CHUNK, xd), jnp.float32),
            pltpu.VMEM((CHUNK, xd), jnp.float32),
            pltpu.VMEM_SHARED((ACC_ROWS, xd), jnp.float32),
            pltpu.SemaphoreType.DMA,
            pltpu.SemaphoreType.DMA,
            pltpu.SemaphoreType.DMA,
            pltpu.SemaphoreType.DMA,
            pltpu.SemaphoreType.DMA,
        ],
    )
    def prop(x_hbm, src_hbm, dst_hbm, out_hbm,
             idx_s, idx_d, rows0, rows1, acc,
             sem0, sem1, semd0, semd1, sem_ld):
        cid = lax.axis_index("c")
        sid = lax.axis_index("s")
        wid = cid * NS + sid

        pltpu.async_copy(src_hbm.at[wid], idx_s, sem_ld)

        _zero_fill(rows0, CHUNK, xd)
        zrows_per_tile = ACC_ROWS // NS
        for t in range(zrows_per_tile // CHUNK):
            pltpu.sync_copy(
                rows0, acc.at[pl.ds(sid * zrows_per_tile + t * CHUNK, CHUNK)])

        pltpu.make_async_copy(src_hbm.at[wid], idx_s, sem_ld).wait()
        plsc.subcore_barrier()

        bufs = (rows0, rows1)
        sems = (sem0, sem1)
        semds = (semd0, semd1)
        for b in range(2):
            pltpu.async_copy(dst_hbm.at[wid, b], idx_d.at[b], semds[b])
            pltpu.async_copy(x_hbm.at[idx_s.at[b]], bufs[b], sems[b])

        def pair_body(p, carry):
            for b in range(2):
                j = 2 * p + b
                pltpu.make_async_copy(x_hbm.at[idx_s.at[j]], bufs[b],
                                      sems[b]).wait()
                pltpu.make_async_copy(dst_hbm.at[wid, j], idx_d.at[b],
                                      semds[b]).wait()
                pltpu.sync_copy(bufs[b], acc.at[idx_d.at[b]], add=True)

                @pl.when(j + 2 < NCHUNK)
                def _():
                    pltpu.async_copy(dst_hbm.at[wid, j + 2], idx_d.at[b],
                                     semds[b])
                    pltpu.async_copy(x_hbm.at[idx_s.at[j + 2]], bufs[b],
                                     sems[b])
            return carry

        lax.fori_loop(0, NCHUNK // 2, pair_body, 0)
        plsc.subcore_barrier()

        orows = 624
        pltpu.sync_copy(acc.at[pl.ds(sid * orows, orows)],
                        out_hbm.at[cid, pl.ds(sid * orows, orows)])

        @pl.when(sid == NS - 1)
        def _():
            pltpu.sync_copy(acc.at[pl.ds(NS * orows, N - NS * orows)],
                            out_hbm.at[cid, pl.ds(NS * orows, N - NS * orows)])

    return prop


_propagate = _make_propagate(D)


_RB = 1000


def _dense_body(acc_ref, cnt_ref, x_ref, wl_ref, bl_ref, wr_ref, o_ref):
    a = acc_ref[0] + acc_ref[1]
    c = cnt_ref[0, :, 0] + cnt_ref[1, :, 0]
    mean = a / jnp.clip(c, 1.0, None)[:, None]
    h = (jnp.dot(mean, wl_ref[...], preferred_element_type=jnp.float32)
         + bl_ref[0][None, :]
         + jnp.dot(x_ref[...], wr_ref[...], preferred_element_type=jnp.float32))
    nrm = jnp.sqrt(jnp.sum(h * h, axis=1, keepdims=True))
    o_ref[...] = h / jnp.clip(nrm, 1e-12, None)


_dense = pl.pallas_call(
    _dense_body,
    grid=(N // _RB,),
    in_specs=[
        pl.BlockSpec((NC, _RB, D), lambda i: (0, i, 0)),
        pl.BlockSpec((NC, _RB, D), lambda i: (0, i, 0)),
        pl.BlockSpec((_RB, D), lambda i: (i, 0)),
        pl.BlockSpec((D, D), lambda i: (0, 0)),
        pl.BlockSpec((1, D), lambda i: (0, 0)),
        pl.BlockSpec((D, D), lambda i: (0, 0)),
    ],
    out_specs=pl.BlockSpec((_RB, D), lambda i: (i, 0)),
    out_shape=jax.ShapeDtypeStruct((N, D), jnp.float32),
)


def _fc_body(x1_ref, x2_ref, x3_ref, w_ref, b_ref, o_ref):
    w = w_ref[...]
    h = (jnp.dot(x1_ref[...], w[0:D], preferred_element_type=jnp.float32)
         + jnp.dot(x2_ref[...], w[D:2 * D], preferred_element_type=jnp.float32)
         + jnp.dot(x3_ref[...], w[2 * D:3 * D], preferred_element_type=jnp.float32)
         + b_ref[0][None, :])
    o_ref[...] = h


_fc = pl.pallas_call(
    _fc_body,
    grid=(N // _RB,),
    in_specs=[
        pl.BlockSpec((_RB, D), lambda i: (i, 0)),
        pl.BlockSpec((_RB, D), lambda i: (i, 0)),
        pl.BlockSpec((_RB, D), lambda i: (i, 0)),
        pl.BlockSpec((3 * D, D), lambda i: (0, 0)),
        pl.BlockSpec((1, D), lambda i: (0, 0)),
    ],
    out_specs=pl.BlockSpec((_RB, D), lambda i: (i, 0)),
    out_shape=jax.ShapeDtypeStruct((N, D), jnp.float32),
)


def _prep_edges(edge_index):
    pad = E_PAD - E
    src = jnp.concatenate([edge_index[0], jnp.zeros((pad,), jnp.int32)])
    dst = jnp.concatenate([edge_index[1], jnp.full((pad,), N, jnp.int32)])
    return src.reshape(NW, NCHUNK, CHUNK), dst.reshape(NW, NCHUNK, CHUNK)


def _sage_layer(x, src3, dst3, cnt, p):
    acc = _propagate(x, src3, dst3)
    return _dense(acc, cnt, x,
                  p["Wl"], p["bl"].reshape(1, D), p["Wr"])


def _two_conv(x, src3, dst3, cnt, p):
    x = _sage_layer(x, src3, dst3, cnt, p["c1"])
    x = _sage_layer(x, src3, dst3, cnt, p["c2"])
    return x


def kernel(features1, edge_index1, features2, edge_index2,
           features3, edge_index3, params):
    e1 = _prep_edges(edge_index1)
    e2 = _prep_edges(edge_index2)
    e3 = _prep_edges(edge_index3)
    ones = jnp.ones((N, D), jnp.float32)
    c1 = _propagate(ones, *e1)
    c2 = _propagate(ones, *e2)
    c3 = _propagate(ones, *e3)

    x1 = _two_conv(features1, *e1, c1, params["enc1"])
    x2 = _two_conv(features2, *e2, c2, params["enc2"])
    x3 = _two_conv(features3, *e3, c3, params["enc3"])

    x = _fc(x1, x2, x3, params["fc"]["W"],
            params["fc"]["b"].reshape(1, D))

    x1_rec = _two_conv(x, *e1, c1, params["dec1"])
    x2_rec = _two_conv(x, *e2, c2, params["dec2"])
    x3_rec = _two_conv(x, *e3, c3, params["dec3"])
    return (x, x1_rec, x2_rec, x3_rec)


# speedup vs baseline: 2.0277x; 1.2089x over previous
import functools

import jax
import jax.numpy as jnp
from jax import lax
from jax.experimental import pallas as pl
from jax.experimental.pallas import tpu as pltpu
from jax.experimental.pallas import tpu_sc as plsc

N = 10000
D = 128
E = 320000

NC = 2
NS = 16
NW = NC * NS
CHUNK = 128
EPW = 10240
NCHUNK = EPW // CHUNK
E_PAD = NW * EPW
ACC_ROWS = 10240
ZR = 64

_mesh = plsc.VectorSubcoreMesh(core_axis_name="c", subcore_axis_name="s")


def _zero_fill(zbuf, rows, cols):
    z = jnp.zeros((16,), jnp.float32)
    for i in range(rows):
        for k in range(cols // 16):
            zbuf[i, pl.ds(16 * k, 16)] = z


def _make_propagate(xd):
    @functools.partial(
        pl.kernel,
        out_type=jax.ShapeDtypeStruct((NC, N, xd), jnp.float32),
        mesh=_mesh,
        scratch_types=[
            pltpu.VMEM((NCHUNK, CHUNK), jnp.int32),
            pltpu.VMEM((2, CHUNK), jnp.int32),
            pltpu.VMEM((CHUNK, xd), jnp.float32),
            pltpu.VMEM((CHUNK, xd), jnp.float32),
            pltpu.VMEM_SHARED((ACC_ROWS, xd), jnp.float32),
            pltpu.SemaphoreType.DMA,
            pltpu.SemaphoreType.DMA,
            pltpu.SemaphoreType.DMA,
            pltpu.SemaphoreType.DMA,
            pltpu.SemaphoreType.DMA,
        ],
    )
    def prop(x_hbm, src_hbm, dst_hbm, out_hbm,
             idx_s, idx_d, rows0, rows1, acc,
             sem0, sem1, semd0, semd1, sem_ld):
        cid = lax.axis_index("c")
        sid = lax.axis_index("s")
        wid = cid * NS + sid

        pltpu.async_copy(src_hbm.at[wid], idx_s, sem_ld)

        _zero_fill(rows0, CHUNK, xd)
        zrows_per_tile = ACC_ROWS // NS
        for t in range(zrows_per_tile // CHUNK):
            pltpu.sync_copy(
                rows0, acc.at[pl.ds(sid * zrows_per_tile + t * CHUNK, CHUNK)])

        pltpu.make_async_copy(src_hbm.at[wid], idx_s, sem_ld).wait()
        plsc.subcore_barrier()

        bufs = (rows0, rows1)
        sems = (sem0, sem1)
        semds = (semd0, semd1)
        for b in range(2):
            pltpu.async_copy(dst_hbm.at[wid, b], idx_d.at[b], semds[b])
            pltpu.async_copy(x_hbm.at[idx_s.at[b]], bufs[b], sems[b])

        def pair_body(p, carry):
            for b in range(2):
                j = 2 * p + b
                pltpu.make_async_copy(x_hbm.at[idx_s.at[j]], bufs[b],
                                      sems[b]).wait()
                pltpu.make_async_copy(dst_hbm.at[wid, j], idx_d.at[b],
                                      semds[b]).wait()
                pltpu.sync_copy(bufs[b], acc.at[idx_d.at[b]], add=True)

                @pl.when(j + 2 < NCHUNK)
                def _():
                    pltpu.async_copy(dst_hbm.at[wid, j + 2], idx_d.at[b],
                                     semds[b])
                    pltpu.async_copy(x_hbm.at[idx_s.at[j + 2]], bufs[b],
                                     sems[b])
            return carry

        lax.fori_loop(0, NCHUNK // 2, pair_body, 0)
        plsc.subcore_barrier()

        orows = 624
        pltpu.sync_copy(acc.at[pl.ds(sid * orows, orows)],
                        out_hbm.at[cid, pl.ds(sid * orows, orows)])

        @pl.when(sid == NS - 1)
        def _():
            pltpu.sync_copy(acc.at[pl.ds(NS * orows, N - NS * orows)],
                            out_hbm.at[cid, pl.ds(NS * orows, N - NS * orows)])

    return prop


_propagate = _make_propagate(D)


@functools.partial(
    pl.kernel,
    out_type=jax.ShapeDtypeStruct((NC, N, D), jnp.float32),
    mesh=_mesh,
    scratch_types=[
        pltpu.VMEM((2, CHUNK), jnp.int32),
        pltpu.VMEM((CHUNK, D), jnp.float32),
        pltpu.VMEM_SHARED((ACC_ROWS, D), jnp.float32),
        pltpu.SemaphoreType.DMA,
        pltpu.SemaphoreType.DMA,
    ],
)
def _count(dst_hbm, out_hbm, idx_d, ones_v, acc, semd0, semd1):
    cid = lax.axis_index("c")
    sid = lax.axis_index("s")
    wid = cid * NS + sid

    _zero_fill(ones_v, CHUNK, D)
    zrows_per_tile = ACC_ROWS // NS
    for t in range(zrows_per_tile // CHUNK):
        pltpu.sync_copy(
            ones_v, acc.at[pl.ds(sid * zrows_per_tile + t * CHUNK, CHUNK)])
    one = jnp.ones((16,), jnp.float32)
    for i in range(CHUNK):
        for k in range(D // 16):
            ones_v[i, pl.ds(16 * k, 16)] = one
    plsc.subcore_barrier()

    semds = (semd0, semd1)
    for b in range(2):
        pltpu.async_copy(dst_hbm.at[wid, b], idx_d.at[b], semds[b])

    def pair_body(p, carry):
        for b in range(2):
            j = 2 * p + b
            pltpu.make_async_copy(dst_hbm.at[wid, j], idx_d.at[b],
                                  semds[b]).wait()
            pltpu.sync_copy(ones_v, acc.at[idx_d.at[b]], add=True)

            @pl.when(j + 2 < NCHUNK)
            def _():
                pltpu.async_copy(dst_hbm.at[wid, j + 2], idx_d.at[b],
                                 semds[b])
        return carry

    lax.fori_loop(0, NCHUNK // 2, pair_body, 0)
    plsc.subcore_barrier()

    orows = 624
    pltpu.sync_copy(acc.at[pl.ds(sid * orows, orows)],
                    out_hbm.at[cid, pl.ds(sid * orows, orows)])

    @pl.when(sid == NS - 1)
    def _():
        pltpu.sync_copy(acc.at[pl.ds(NS * orows, N - NS * orows)],
                        out_hbm.at[cid, pl.ds(NS * orows, N - NS * orows)])


_RB = 1000


def _dense_body(acc_ref, cnt_ref, x_ref, wl_ref, bl_ref, wr_ref, o_ref):
    a = acc_ref[0] + acc_ref[1]
    c = cnt_ref[0, :, 0] + cnt_ref[1, :, 0]
    mean = a / jnp.clip(c, 1.0, None)[:, None]
    h = (jnp.dot(mean, wl_ref[...], preferred_element_type=jnp.float32)
         + bl_ref[0][None, :]
         + jnp.dot(x_ref[...], wr_ref[...], preferred_element_type=jnp.float32))
    nrm = jnp.sqrt(jnp.sum(h * h, axis=1, keepdims=True))
    o_ref[...] = h / jnp.clip(nrm, 1e-12, None)


_dense = pl.pallas_call(
    _dense_body,
    grid=(N // _RB,),
    in_specs=[
        pl.BlockSpec((NC, _RB, D), lambda i: (0, i, 0)),
        pl.BlockSpec((NC, _RB, D), lambda i: (0, i, 0)),
        pl.BlockSpec((_RB, D), lambda i: (i, 0)),
        pl.BlockSpec((D, D), lambda i: (0, 0)),
        pl.BlockSpec((1, D), lambda i: (0, 0)),
        pl.BlockSpec((D, D), lambda i: (0, 0)),
    ],
    out_specs=pl.BlockSpec((_RB, D), lambda i: (i, 0)),
    out_shape=jax.ShapeDtypeStruct((N, D), jnp.float32),
)


def _fc_body(x1_ref, x2_ref, x3_ref, w_ref, b_ref, o_ref):
    w = w_ref[...]
    h = (jnp.dot(x1_ref[...], w[0:D], preferred_element_type=jnp.float32)
         + jnp.dot(x2_ref[...], w[D:2 * D], preferred_element_type=jnp.float32)
         + jnp.dot(x3_ref[...], w[2 * D:3 * D], preferred_element_type=jnp.float32)
         + b_ref[0][None, :])
    o_ref[...] = h


_fc = pl.pallas_call(
    _fc_body,
    grid=(N // _RB,),
    in_specs=[
        pl.BlockSpec((_RB, D), lambda i: (i, 0)),
        pl.BlockSpec((_RB, D), lambda i: (i, 0)),
        pl.BlockSpec((_RB, D), lambda i: (i, 0)),
        pl.BlockSpec((3 * D, D), lambda i: (0, 0)),
        pl.BlockSpec((1, D), lambda i: (0, 0)),
    ],
    out_specs=pl.BlockSpec((_RB, D), lambda i: (i, 0)),
    out_shape=jax.ShapeDtypeStruct((N, D), jnp.float32),
)


def _prep_edges(edge_index):
    pad = E_PAD - E
    src = jnp.concatenate([edge_index[0], jnp.zeros((pad,), jnp.int32)])
    dst = jnp.concatenate([edge_index[1], jnp.full((pad,), N, jnp.int32)])
    return src.reshape(NW, NCHUNK, CHUNK), dst.reshape(NW, NCHUNK, CHUNK)


def _sage_layer(x, src3, dst3, cnt, p):
    acc = _propagate(x, src3, dst3)
    return _dense(acc, cnt, x,
                  p["Wl"], p["bl"].reshape(1, D), p["Wr"])


def _two_conv(x, src3, dst3, cnt, p):
    x = _sage_layer(x, src3, dst3, cnt, p["c1"])
    x = _sage_layer(x, src3, dst3, cnt, p["c2"])
    return x


def kernel(features1, edge_index1, features2, edge_index2,
           features3, edge_index3, params):
    e1 = _prep_edges(edge_index1)
    e2 = _prep_edges(edge_index2)
    e3 = _prep_edges(edge_index3)
    c1 = _count(e1[1])
    c2 = _count(e2[1])
    c3 = _count(e3[1])

    x1 = _two_conv(features1, *e1, c1, params["enc1"])
    x2 = _two_conv(features2, *e2, c2, params["enc2"])
    x3 = _two_conv(features3, *e3, c3, params["enc3"])

    x = _fc(x1, x2, x3, params["fc"]["W"],
            params["fc"]["b"].reshape(1, D))

    x1_rec = _two_conv(x, *e1, c1, params["dec1"])
    x2_rec = _two_conv(x, *e2, c2, params["dec2"])
    x3_rec = _two_conv(x, *e3, c3, params["dec3"])
    return (x, x1_rec, x2_rec, x3_rec)
